# Initial kernel scaffold; baseline (speedup 1.0000x reference)
import jax
import jax.numpy as jnp
from jax.experimental import pallas as pl


def kernel(x, edge_index, edge_index_sim, batch, W1, b1, Wg, bg, Ws, bs, Ww, bw):
    raise NotImplementedError("write your pallas kernel here")


# speedup vs baseline: 11.7231x; 11.7231x over previous
import jax
import jax.numpy as jnp
from jax import lax
from jax.experimental import pallas as pl
from jax.experimental.pallas import tpu as pltpu
from jax.experimental.pallas import tpu_sc as plsc

N = 50000
E = 800000
H = 64
NG = 64
R = 1000
GRID = N // R

NS = 16
EPT = E // NS
CH = 80
NB = 5
NGRP = EPT // (CH * NB)
RPT = 3128
NP = NS * RPT

_f32 = jnp.float32


def _mesh():
    return plsc.VectorSubcoreMesh(core_axis_name="c", subcore_axis_name="s")


def _sc_deg_body(dstB, out, acc, zbuf, ones, d0, d1, d2, d3, d4,
                 si0, si1, si2, si3, si4, ss0, ss1, ss2, ss3, ss4):
    c = lax.axis_index("c")
    t = lax.axis_index("s")
    dbufs = (d0, d1, d2, d3, d4)
    sis = (si0, si1, si2, si3, si4)
    sss = (ss0, ss1, ss2, ss3, ss4)

    def fill(i, _):
        zbuf[i, pl.ds(0, 16)] = jnp.zeros((16,), _f32)
        return 0
    lax.fori_loop(0, RPT, fill, 0)

    def fill1(i, _):
        ones[i, pl.ds(0, 16)] = jnp.ones((16,), _f32)
        return 0
    lax.fori_loop(0, CH, fill1, 0)

    pltpu.sync_copy(zbuf, acc.at[pl.ds(t * RPT, RPT)])
    plsc.subcore_barrier()

    base = c * E + t * EPT

    def grp(g, _):
        j0 = g * NB
        ids = []
        for b in range(NB):
            off = base + (j0 + b) * CH
            ids.append(pltpu.async_copy(dstB.at[pl.ds(off, CH)],
                                        dbufs[b], sis[b]))
        sds = []
        for b in range(NB):
            ids[b].wait()
            sds.append(pltpu.async_copy(ones, acc.at[dbufs[b]], sss[b],
                                        add=True))
        for b in range(NB):
            sds[b].wait()
        return 0
    lax.fori_loop(0, NGRP, grp, 0)
    plsc.subcore_barrier()

    @pl.when(t == 0)
    def _():
        pltpu.sync_copy(acc, out.at[c])


_sc_deg = pl.kernel(
    _sc_deg_body,
    out_type=jax.ShapeDtypeStruct((2, NP, 16), _f32),
    mesh=_mesh(),
    compiler_params=pltpu.CompilerParams(use_tc_tiling_on_sc=False),
    scratch_types=[
        pltpu.VMEM_SHARED((NP, 16), _f32),
        pltpu.VMEM((RPT, 16), _f32),
        pltpu.VMEM((CH, 16), _f32),
    ] + [pltpu.VMEM((CH,), jnp.int32) for _ in range(NB)]
      + [pltpu.SemaphoreType.DMA for _ in range(2 * NB)],
)


def _sc_agg_body(y4, gidx, dst, out, acc, zbuf,
                 s0, s1, s2, s3, s4, d0, d1, d2, d3, d4,
                 r0b, r1b, r2b, r3b, r4b,
                 si0, si1, si2, si3, si4, sg0, sg1, sg2, sg3, sg4,
                 ss0, ss1, ss2, ss3, ss4):
    c = lax.axis_index("c")
    t = lax.axis_index("s")
    sbufs = (s0, s1, s2, s3, s4)
    dbufs = (d0, d1, d2, d3, d4)
    rbufs = (r0b, r1b, r2b, r3b, r4b)
    sis = (si0, si1, si2, si3, si4)
    sgs = (sg0, sg1, sg2, sg3, sg4)
    sss = (ss0, ss1, ss2, ss3, ss4)

    def fill(i, _):
        zbuf[i, pl.ds(0, 16)] = jnp.zeros((16,), _f32)
        return 0
    lax.fori_loop(0, RPT, fill, 0)

    for p in range(2):
        q = 2 * c + p
        pltpu.sync_copy(zbuf, acc.at[pl.ds(t * RPT, RPT)])
        plsc.subcore_barrier()

        base = q * E + t * EPT

        def grp(g, _):
            j0 = g * NB
            ids = []
            for b in range(NB):
                off = base + (j0 + b) * CH
                i1 = pltpu.async_copy(gidx.at[pl.ds(off, CH)], sbufs[b],
                                      sis[b])
                i2 = pltpu.async_copy(dst.at[pl.ds(t * EPT + (j0 + b) * CH,
                                                   CH)], dbufs[b], sis[b])
                ids.append((i1, i2))
            gds = []
            for b in range(NB):
                ids[b][0].wait()
                ids[b][1].wait()
                gds.append(pltpu.async_copy(y4.at[sbufs[b]], rbufs[b],
                                            sgs[b]))
            sds = []
            for b in range(NB):
                gds[b].wait()
                sds.append(pltpu.async_copy(rbufs[b], acc.at[dbufs[b]],
                                            sss[b], add=True))
            for b in range(NB):
                sds[b].wait()
            return 0
        lax.fori_loop(0, NGRP, grp, 0)
        plsc.subcore_barrier()

        @pl.when(t == 0)
        def _():
            pltpu.sync_copy(acc, out.at[q])
        plsc.subcore_barrier()


_sc_agg = pl.kernel(
    _sc_agg_body,
    out_type=jax.ShapeDtypeStruct((4, NP, 16), _f32),
    mesh=_mesh(),
    compiler_params=pltpu.CompilerParams(use_tc_tiling_on_sc=False),
    scratch_types=[
        pltpu.VMEM_SHARED((NP, 16), _f32),
        pltpu.VMEM((RPT, 16), _f32),
    ] + [pltpu.VMEM((CH,), jnp.int32) for _ in range(2 * NB)]
      + [pltpu.VMEM((CH, 16), _f32) for _ in range(NB)]
      + [pltpu.SemaphoreType.DMA for _ in range(3 * NB)],
)


def _gate_pre(h, dg, ds, Wg, Ws, Wwt, bw):
    yg = dg * jnp.dot(h, Wg, preferred_element_type=_f32)
    ys = ds * jnp.dot(h, Ws, preferred_element_type=_f32)
    s = jax.nn.sigmoid(jnp.sum(h * Wwt, axis=1, keepdims=True) + bw)
    return yg, ys, s


def _store_quarters(ref, y):
    for q in range(4):
        ref[q, 0] = y[:, 16 * q:16 * (q + 1)]


def _tc_init_body(x_ref, degg_ref, degs_ref, W1_ref, b1_ref, Wg_ref, Ws_ref,
                  Wwt_ref, bw_ref, yg_ref, ys_ref, s_ref, dg_ref, ds_ref):
    h = x_ref[...] * W1_ref[...] + b1_ref[...]
    dg = lax.rsqrt(degg_ref[:, 0:1] + 1.0)
    ds = lax.rsqrt(degs_ref[:, 0:1] + 1.0)
    yg, ys, s = _gate_pre(h, dg, ds, Wg_ref[...], Ws_ref[...], Wwt_ref[...],
                          bw_ref[...])
    _store_quarters(yg_ref, yg)
    _store_quarters(ys_ref, ys)
    s_ref[...] = s
    dg_ref[...] = dg
    ds_ref[...] = ds


def _combine(ag, asm, yg, ys, s_ref, dg_ref, ds_ref, bg_ref, bs_ref):
    dg = dg_ref[...]
    ds = ds_ref[...]
    agg_g = jnp.concatenate([r[...] for r in ag], axis=1)
    agg_s = jnp.concatenate([r[...] for r in asm], axis=1)
    y_g = jnp.concatenate([r[...] for r in yg], axis=1)
    y_s = jnp.concatenate([r[...] for r in ys], axis=1)
    xg = jnp.maximum(dg * (agg_g + y_g) + bg_ref[...], 0.0)
    xs = jnp.maximum(ds * (agg_s + y_s) + bs_ref[...], 0.0)
    s = s_ref[...]
    return s * xg + (1.0 - s) * xs, dg, ds


def _tc_mid_body(*refs):
    (ag0, ag1, ag2, ag3, as0, as1, as2, as3,
     yg0, yg1, yg2, yg3, ys0, ys1, ys2, ys3,
     s_ref, dg_ref, ds_ref, bg_ref, bs_ref,
     Wg_ref, Ws_ref, Wwt_ref, bw_ref, yg_ref, ys_ref, so_ref) = refs
    h, dg, ds = _combine((ag0, ag1, ag2, ag3), (as0, as1, as2, as3),
                         (yg0, yg1, yg2, yg3), (ys0, ys1, ys2, ys3),
                         s_ref, dg_ref, ds_ref, bg_ref, bs_ref)
    yg, ys, s = _gate_pre(h, dg, ds, Wg_ref[...], Ws_ref[...], Wwt_ref[...],
                          bw_ref[...])
    _store_quarters(yg_ref, yg)
    _store_quarters(ys_ref, ys)
    so_ref[...] = s


def _tc_fin_body(*refs):
    (ag0, ag1, ag2, ag3, as0, as1, as2, as3,
     yg0, yg1, yg2, yg3, ys0, ys1, ys2, ys3,
     s_ref, dg_ref, ds_ref, bg_ref, bs_ref, batch_ref, out_ref) = refs
    h, _, _ = _combine((ag0, ag1, ag2, ag3), (as0, as1, as2, as3),
                       (yg0, yg1, yg2, yg3), (ys0, ys1, ys2, ys3),
                       s_ref, dg_ref, ds_ref, bg_ref, bs_ref)
    i = pl.program_id(0)

    @pl.when(i == 0)
    def _():
        out_ref[...] = jnp.full((NG, H), -jnp.inf, _f32)

    bb = batch_ref[0, 0, :].reshape(R, 1)
    glo = jnp.min(bb)
    ghi = jnp.max(bb)

    def gbody(g, _):
        m = jnp.max(jnp.where(bb == g, h, -jnp.inf), axis=0)
        out_ref[pl.ds(g, 1), :] = jnp.maximum(out_ref[pl.ds(g, 1), :],
                                              m[None, :])
        return 0
    lax.fori_loop(glo, ghi + 1, gbody, 0)


def _row(i):
    return (i, 0)


_BS_X = pl.BlockSpec((R, 1), _row)
_BS_DEG = pl.BlockSpec((R, 16), _row)
_BS_16 = pl.BlockSpec((R, 16), _row)
_BS_1 = pl.BlockSpec((R, 1), _row)
_BS_Y4 = pl.BlockSpec((4, 1, R, 16), lambda i: (0, i, 0, 0))
_BS_W = pl.BlockSpec((H, H), lambda i: (0, 0))
_BS_W1 = pl.BlockSpec((1, H), lambda i: (0, 0))
_BS_BW = pl.BlockSpec((1, 1), lambda i: (0, 0))
_BS_BATCH = pl.BlockSpec((1, 1, R), lambda i: (i, 0, 0))
_BS_OUT = pl.BlockSpec((NG, H), lambda i: (0, 0))

_y4_shape = jax.ShapeDtypeStruct((4, GRID, R, 16), _f32)
_s_shape = jax.ShapeDtypeStruct((N, 1), _f32)

_tc_init = pl.pallas_call(
    _tc_init_body,
    grid=(GRID,),
    in_specs=[_BS_X, _BS_DEG, _BS_DEG, _BS_W1, _BS_W1, _BS_W, _BS_W,
              _BS_W1, _BS_BW],
    out_specs=[_BS_Y4, _BS_Y4, _BS_1, _BS_1, _BS_1],
    out_shape=[_y4_shape, _y4_shape, _s_shape, _s_shape, _s_shape],
)

_tc_mid = pl.pallas_call(
    _tc_mid_body,
    grid=(GRID,),
    in_specs=[_BS_16] * 16 + [_BS_1] * 3 + [_BS_W1, _BS_W1] +
             [_BS_W, _BS_W, _BS_W1, _BS_BW],
    out_specs=[_BS_Y4, _BS_Y4, _BS_1],
    out_shape=[_y4_shape, _y4_shape, _s_shape],
)

_tc_fin = pl.pallas_call(
    _tc_fin_body,
    grid=(GRID,),
    in_specs=[_BS_16] * 16 + [_BS_1] * 3 + [_BS_W1, _BS_W1] + [_BS_BATCH],
    out_specs=_BS_OUT,
    out_shape=jax.ShapeDtypeStruct((NG, H), _f32),
)


def _quarters(a4):
    return tuple(a4[q, :N] for q in range(4))


def kernel(x, edge_index, edge_index_sim, batch, W1, b1, Wg, bg, Ws, bs,
           Ww, bw):
    ei = edge_index.astype(jnp.int32)
    es = edge_index_sim.astype(jnp.int32)
    src_g, dst_g = ei[0], ei[1]
    src_s, dst_s = es[0], es[1]
    dstB = jnp.concatenate([dst_g, dst_s])
    gidx_g = jnp.concatenate([src_g, src_g + N, src_g + 2 * N,
                              src_g + 3 * N])
    gidx_s = jnp.concatenate([src_s, src_s + N, src_s + 2 * N,
                              src_s + 3 * N])
    batch3 = batch.astype(jnp.int32).reshape(GRID, 1, R)

    deg2 = _sc_deg(dstB)
    degg = deg2[0, :N]
    degs = deg2[1, :N]
    b1r = b1.reshape(1, H)
    yg4, ys4, s, dg, ds = _tc_init(
        x, degg, degs, W1, b1r, Wg[0], Ws[0], Ww[0].reshape(1, H),
        bw[0].reshape(1, 1))

    for i in range(3):
        aggg = _sc_agg(yg4.reshape(4 * N, 16), gidx_g, dst_g)
        aggs = _sc_agg(ys4.reshape(4 * N, 16), gidx_s, dst_s)
        ygq = tuple(yg4.reshape(4, N, 16)[q] for q in range(4))
        ysq = tuple(ys4.reshape(4, N, 16)[q] for q in range(4))
        bgr = bg[i].reshape(1, H)
        bsr = bs[i].reshape(1, H)
        if i < 2:
            yg4, ys4, s = _tc_mid(
                *_quarters(aggg), *_quarters(aggs), *ygq, *ysq,
                s, dg, ds, bgr, bsr,
                Wg[i + 1], Ws[i + 1], Ww[i + 1].reshape(1, H),
                bw[i + 1].reshape(1, 1))
        else:
            out = _tc_fin(
                *_quarters(aggg), *_quarters(aggs), *ygq, *ysq,
                s, dg, ds, bgr, bsr, batch3)
    return out


# speedup vs baseline: 14.6828x; 1.2525x over previous
import jax
import jax.numpy as jnp
from jax import lax
from jax.experimental import pallas as pl
from jax.experimental.pallas import tpu as pltpu
from jax.experimental.pallas import tpu_sc as plsc

N = 50000
E = 800000
H = 64
NG = 64
R = 1000
GRID = N // R

NS = 16
EPT = E // NS
CH = 80
NB = 20
NGRP = 31
NTAIL = 5
RPT = 3128
NP = NS * RPT

_f32 = jnp.float32


def _mesh():
    return plsc.VectorSubcoreMesh(core_axis_name="c", subcore_axis_name="s")


def _sc_deg_body(dstB, zeros_hbm, out, acc, ones, *rest):
    c = lax.axis_index("c")
    t = lax.axis_index("s")
    dbufs = rest[:NB]
    sem_i, sem_s = rest[NB], rest[NB + 1]

    def fill1(i, _):
        ones[i, pl.ds(0, 16)] = jnp.ones((16,), _f32)
        return 0
    lax.fori_loop(0, CH, fill1, 0)

    pltpu.sync_copy(zeros_hbm.at[pl.ds(t * RPT, RPT)],
                    acc.at[pl.ds(t * RPT, RPT)])
    plsc.subcore_barrier()

    base = c * E + t * EPT

    def stage(j0, nb):
        ids = []
        for b in range(nb):
            off = base + j0 * CH + b * CH
            ids.append(pltpu.async_copy(dstB.at[pl.ds(off, CH)],
                                        dbufs[b], sem_i))
        for b in range(nb):
            ids[b].wait()
        sds = []
        for b in range(nb):
            sds.append(pltpu.async_copy(ones, acc.at[dbufs[b]], sem_s,
                                        add=True))
        for b in range(nb):
            sds[b].wait()

    def grp(g, _):
        stage(g * NB, NB)
        return 0
    lax.fori_loop(0, NGRP, grp, 0)
    stage(NGRP * NB, NTAIL)
    plsc.subcore_barrier()

    @pl.when(t == 0)
    def _():
        pltpu.sync_copy(acc, out.at[c])


_sc_deg = pl.kernel(
    _sc_deg_body,
    out_type=jax.ShapeDtypeStruct((2, NP, 16), _f32),
    mesh=_mesh(),
    compiler_params=pltpu.CompilerParams(use_tc_tiling_on_sc=False),
    scratch_types=[
        pltpu.VMEM_SHARED((NP, 16), _f32),
        pltpu.VMEM((CH, 16), _f32),
    ] + [pltpu.VMEM((CH,), jnp.int32) for _ in range(NB)]
      + [pltpu.SemaphoreType.DMA for _ in range(2)],
)


def _sc_agg_body(y4, gidx, dst, zeros_hbm, out, acc, *rest):
    c = lax.axis_index("c")
    t = lax.axis_index("s")
    sbufs = rest[:NB]
    dbufs = rest[NB:2 * NB]
    rbufs = rest[2 * NB:3 * NB]
    sem_i, sem_g, sem_s = rest[3 * NB:3 * NB + 3]

    for p in range(2):
        q = 2 * c + p
        pltpu.sync_copy(zeros_hbm.at[pl.ds(t * RPT, RPT)],
                        acc.at[pl.ds(t * RPT, RPT)])
        plsc.subcore_barrier()

        base = q * E + t * EPT
        dbase = t * EPT

        def stage(j0, nb):
            ids = []
            for b in range(nb):
                i1 = pltpu.async_copy(gidx.at[pl.ds(base + j0 * CH + b * CH,
                                                    CH)], sbufs[b], sem_i)
                i2 = pltpu.async_copy(dst.at[pl.ds(dbase + j0 * CH + b * CH,
                                                   CH)], dbufs[b], sem_i)
                ids.append((i1, i2))
            for b in range(nb):
                ids[b][0].wait()
                ids[b][1].wait()
            gds = []
            for b in range(nb):
                gds.append(pltpu.async_copy(y4.at[sbufs[b]], rbufs[b],
                                            sem_g))
            for b in range(nb):
                gds[b].wait()
            sds = []
            for b in range(nb):
                sds.append(pltpu.async_copy(rbufs[b], acc.at[dbufs[b]],
                                            sem_s, add=True))
            for b in range(nb):
                sds[b].wait()

        def grp(g, _):
            stage(g * NB, NB)
            return 0
        lax.fori_loop(0, NGRP, grp, 0)
        stage(NGRP * NB, NTAIL)
        plsc.subcore_barrier()

        @pl.when(t == 0)
        def _():
            pltpu.sync_copy(acc, out.at[q])
        plsc.subcore_barrier()


_sc_agg = pl.kernel(
    _sc_agg_body,
    out_type=jax.ShapeDtypeStruct((4, NP, 16), _f32),
    mesh=_mesh(),
    compiler_params=pltpu.CompilerParams(use_tc_tiling_on_sc=False),
    scratch_types=[
        pltpu.VMEM_SHARED((NP, 16), _f32),
    ] + [pltpu.VMEM((CH,), jnp.int32) for _ in range(2 * NB)]
      + [pltpu.VMEM((CH, 16), _f32) for _ in range(NB)]
      + [pltpu.SemaphoreType.DMA for _ in range(3)],
)


def _gate_pre(h, dg, ds, Wg, Ws, Wwt, bw):
    yg = dg * jnp.dot(h, Wg, preferred_element_type=_f32)
    ys = ds * jnp.dot(h, Ws, preferred_element_type=_f32)
    s = jax.nn.sigmoid(jnp.sum(h * Wwt, axis=1, keepdims=True) + bw)
    return yg, ys, s


def _store_quarters(ref, y):
    for q in range(4):
        ref[q, 0] = y[:, 16 * q:16 * (q + 1)]


def _tc_init_body(x_ref, degg_ref, degs_ref, W1_ref, b1_ref, Wg_ref, Ws_ref,
                  Wwt_ref, bw_ref, yg_ref, ys_ref, s_ref, dg_ref, ds_ref):
    h = x_ref[...] * W1_ref[...] + b1_ref[...]
    dg = lax.rsqrt(degg_ref[:, 0:1] + 1.0)
    ds = lax.rsqrt(degs_ref[:, 0:1] + 1.0)
    yg, ys, s = _gate_pre(h, dg, ds, Wg_ref[...], Ws_ref[...], Wwt_ref[...],
                          bw_ref[...])
    _store_quarters(yg_ref, yg)
    _store_quarters(ys_ref, ys)
    s_ref[...] = s
    dg_ref[...] = dg
    ds_ref[...] = ds


def _combine(ag, asm, yg, ys, s_ref, dg_ref, ds_ref, bg_ref, bs_ref):
    dg = dg_ref[...]
    ds = ds_ref[...]
    agg_g = jnp.concatenate([r[...] for r in ag], axis=1)
    agg_s = jnp.concatenate([r[...] for r in asm], axis=1)
    y_g = jnp.concatenate([r[...] for r in yg], axis=1)
    y_s = jnp.concatenate([r[...] for r in ys], axis=1)
    xg = jnp.maximum(dg * (agg_g + y_g) + bg_ref[...], 0.0)
    xs = jnp.maximum(ds * (agg_s + y_s) + bs_ref[...], 0.0)
    s = s_ref[...]
    return s * xg + (1.0 - s) * xs, dg, ds


def _tc_mid_body(*refs):
    (ag0, ag1, ag2, ag3, as0, as1, as2, as3,
     yg0, yg1, yg2, yg3, ys0, ys1, ys2, ys3,
     s_ref, dg_ref, ds_ref, bg_ref, bs_ref,
     Wg_ref, Ws_ref, Wwt_ref, bw_ref, yg_ref, ys_ref, so_ref) = refs
    h, dg, ds = _combine((ag0, ag1, ag2, ag3), (as0, as1, as2, as3),
                         (yg0, yg1, yg2, yg3), (ys0, ys1, ys2, ys3),
                         s_ref, dg_ref, ds_ref, bg_ref, bs_ref)
    yg, ys, s = _gate_pre(h, dg, ds, Wg_ref[...], Ws_ref[...], Wwt_ref[...],
                          bw_ref[...])
    _store_quarters(yg_ref, yg)
    _store_quarters(ys_ref, ys)
    so_ref[...] = s


def _tc_fin_body(*refs):
    (ag0, ag1, ag2, ag3, as0, as1, as2, as3,
     yg0, yg1, yg2, yg3, ys0, ys1, ys2, ys3,
     s_ref, dg_ref, ds_ref, bg_ref, bs_ref, batch_ref, out_ref) = refs
    h, _, _ = _combine((ag0, ag1, ag2, ag3), (as0, as1, as2, as3),
                       (yg0, yg1, yg2, yg3), (ys0, ys1, ys2, ys3),
                       s_ref, dg_ref, ds_ref, bg_ref, bs_ref)
    i = pl.program_id(0)

    @pl.when(i == 0)
    def _():
        out_ref[...] = jnp.full((NG, H), -jnp.inf, _f32)

    bb = batch_ref[0, 0, :].reshape(R, 1)
    glo = jnp.min(bb)
    ghi = jnp.max(bb)

    def gbody(g, _):
        m = jnp.max(jnp.where(bb == g, h, -jnp.inf), axis=0)
        out_ref[pl.ds(g, 1), :] = jnp.maximum(out_ref[pl.ds(g, 1), :],
                                              m[None, :])
        return 0
    lax.fori_loop(glo, ghi + 1, gbody, 0)


def _row(i):
    return (i, 0)


_BS_X = pl.BlockSpec((R, 1), _row)
_BS_DEG = pl.BlockSpec((R, 16), _row)
_BS_16 = pl.BlockSpec((R, 16), _row)
_BS_1 = pl.BlockSpec((R, 1), _row)
_BS_Y4 = pl.BlockSpec((4, 1, R, 16), lambda i: (0, i, 0, 0))
_BS_W = pl.BlockSpec((H, H), lambda i: (0, 0))
_BS_W1 = pl.BlockSpec((1, H), lambda i: (0, 0))
_BS_BW = pl.BlockSpec((1, 1), lambda i: (0, 0))
_BS_BATCH = pl.BlockSpec((1, 1, R), lambda i: (i, 0, 0))
_BS_OUT = pl.BlockSpec((NG, H), lambda i: (0, 0))

_y4_shape = jax.ShapeDtypeStruct((4, GRID, R, 16), _f32)
_s_shape = jax.ShapeDtypeStruct((N, 1), _f32)

_tc_init = pl.pallas_call(
    _tc_init_body,
    grid=(GRID,),
    in_specs=[_BS_X, _BS_DEG, _BS_DEG, _BS_W1, _BS_W1, _BS_W, _BS_W,
              _BS_W1, _BS_BW],
    out_specs=[_BS_Y4, _BS_Y4, _BS_1, _BS_1, _BS_1],
    out_shape=[_y4_shape, _y4_shape, _s_shape, _s_shape, _s_shape],
)

_tc_mid = pl.pallas_call(
    _tc_mid_body,
    grid=(GRID,),
    in_specs=[_BS_16] * 16 + [_BS_1] * 3 + [_BS_W1, _BS_W1] +
             [_BS_W, _BS_W, _BS_W1, _BS_BW],
    out_specs=[_BS_Y4, _BS_Y4, _BS_1],
    out_shape=[_y4_shape, _y4_shape, _s_shape],
)

_tc_fin = pl.pallas_call(
    _tc_fin_body,
    grid=(GRID,),
    in_specs=[_BS_16] * 16 + [_BS_1] * 3 + [_BS_W1, _BS_W1] + [_BS_BATCH],
    out_specs=_BS_OUT,
    out_shape=jax.ShapeDtypeStruct((NG, H), _f32),
)


def _quarters(a4):
    return tuple(a4[q, :N] for q in range(4))


def kernel(x, edge_index, edge_index_sim, batch, W1, b1, Wg, bg, Ws, bs,
           Ww, bw):
    ei = edge_index.astype(jnp.int32)
    es = edge_index_sim.astype(jnp.int32)
    src_g, dst_g = ei[0], ei[1]
    src_s, dst_s = es[0], es[1]
    dstB = jnp.concatenate([dst_g, dst_s])
    gidx_g = jnp.concatenate([src_g, src_g + N, src_g + 2 * N,
                              src_g + 3 * N])
    gidx_s = jnp.concatenate([src_s, src_s + N, src_s + 2 * N,
                              src_s + 3 * N])
    batch3 = batch.astype(jnp.int32).reshape(GRID, 1, R)

    zeros16 = jnp.zeros((NP, 16), _f32)
    deg2 = _sc_deg(dstB, zeros16)
    degg = deg2[0, :N]
    degs = deg2[1, :N]
    b1r = b1.reshape(1, H)
    yg4, ys4, s, dg, ds = _tc_init(
        x, degg, degs, W1, b1r, Wg[0], Ws[0], Ww[0].reshape(1, H),
        bw[0].reshape(1, 1))

    for i in range(3):
        aggg = _sc_agg(yg4.reshape(4 * N, 16), gidx_g, dst_g, zeros16)
        aggs = _sc_agg(ys4.reshape(4 * N, 16), gidx_s, dst_s, zeros16)
        ygq = tuple(yg4.reshape(4, N, 16)[q] for q in range(4))
        ysq = tuple(ys4.reshape(4, N, 16)[q] for q in range(4))
        bgr = bg[i].reshape(1, H)
        bsr = bs[i].reshape(1, H)
        if i < 2:
            yg4, ys4, s = _tc_mid(
                *_quarters(aggg), *_quarters(aggs), *ygq, *ysq,
                s, dg, ds, bgr, bsr,
                Wg[i + 1], Ws[i + 1], Ww[i + 1].reshape(1, H),
                bw[i + 1].reshape(1, 1))
        else:
            out = _tc_fin(
                *_quarters(aggg), *_quarters(aggs), *ygq, *ysq,
                s, dg, ds, bgr, bsr, batch3)
    return out


# speedup vs baseline: 14.7937x; 1.0076x over previous
import jax
import jax.numpy as jnp
from jax import lax
from jax.experimental import pallas as pl
from jax.experimental.pallas import tpu as pltpu
from jax.experimental.pallas import tpu_sc as plsc

N = 50000
E = 800000
H = 64
NG = 64
R = 1000
GRID = N // R

NS = 16
EPT = E // NS
CH = 400
NB = 4
NGRP = 31
NTAIL = 1
RPT = 3128
NP = NS * RPT

_f32 = jnp.float32


def _mesh():
    return plsc.VectorSubcoreMesh(core_axis_name="c", subcore_axis_name="s")


def _sc_deg_body(dstB, zeros_hbm, out, acc, ones, *rest):
    c = lax.axis_index("c")
    t = lax.axis_index("s")
    dbufs = rest[:NB]
    sem_i, sem_s = rest[NB], rest[NB + 1]

    def fill1(i, _):
        ones[i, pl.ds(0, 16)] = jnp.ones((16,), _f32)
        return 0
    lax.fori_loop(0, CH, fill1, 0)

    pltpu.sync_copy(zeros_hbm.at[pl.ds(t * RPT, RPT)],
                    acc.at[pl.ds(t * RPT, RPT)])
    plsc.subcore_barrier()

    base = c * E + t * EPT

    def stage(j0, nb):
        ids = []
        for b in range(nb):
            off = base + j0 * CH + b * CH
            ids.append(pltpu.async_copy(dstB.at[pl.ds(off, CH)],
                                        dbufs[b], sem_i))
        for b in range(nb):
            ids[b].wait()
        sds = []
        for b in range(nb):
            sds.append(pltpu.async_copy(ones, acc.at[dbufs[b]], sem_s,
                                        add=True))
        for b in range(nb):
            sds[b].wait()

    def grp(g, _):
        stage(g * NB, NB)
        return 0
    lax.fori_loop(0, NGRP, grp, 0)
    stage(NGRP * NB, NTAIL)
    plsc.subcore_barrier()

    @pl.when(t == 0)
    def _():
        pltpu.sync_copy(acc, out.at[c])


_sc_deg = pl.kernel(
    _sc_deg_body,
    out_type=jax.ShapeDtypeStruct((2, NP, 16), _f32),
    mesh=_mesh(),
    compiler_params=pltpu.CompilerParams(use_tc_tiling_on_sc=False),
    scratch_types=[
        pltpu.VMEM_SHARED((NP, 16), _f32),
        pltpu.VMEM((CH, 16), _f32),
    ] + [pltpu.VMEM((CH,), jnp.int32) for _ in range(NB)]
      + [pltpu.SemaphoreType.DMA for _ in range(2)],
)


def _sc_agg_body(y4, gidx, dst, zeros_hbm, out, acc, *rest):
    c = lax.axis_index("c")
    t = lax.axis_index("s")
    sbufs = rest[:NB]
    dbufs = rest[NB:2 * NB]
    rbufs = rest[2 * NB:3 * NB]
    sem_i, sem_g, sem_s = rest[3 * NB:3 * NB + 3]

    for p in range(2):
        q = 2 * c + p
        pltpu.sync_copy(zeros_hbm.at[pl.ds(t * RPT, RPT)],
                        acc.at[pl.ds(t * RPT, RPT)])
        plsc.subcore_barrier()

        base = q * E + t * EPT
        dbase = t * EPT

        def stage(j0, nb):
            ids = []
            for b in range(nb):
                i1 = pltpu.async_copy(gidx.at[pl.ds(base + j0 * CH + b * CH,
                                                    CH)], sbufs[b], sem_i)
                i2 = pltpu.async_copy(dst.at[pl.ds(dbase + j0 * CH + b * CH,
                                                   CH)], dbufs[b], sem_i)
                ids.append((i1, i2))
            for b in range(nb):
                ids[b][0].wait()
                ids[b][1].wait()
            gds = []
            for b in range(nb):
                gds.append(pltpu.async_copy(y4.at[sbufs[b]], rbufs[b],
                                            sem_g))
            for b in range(nb):
                gds[b].wait()
            sds = []
            for b in range(nb):
                sds.append(pltpu.async_copy(rbufs[b], acc.at[dbufs[b]],
                                            sem_s, add=True))
            for b in range(nb):
                sds[b].wait()

        def grp(g, _):
            stage(g * NB, NB)
            return 0
        lax.fori_loop(0, NGRP, grp, 0)
        stage(NGRP * NB, NTAIL)
        plsc.subcore_barrier()

        @pl.when(t == 0)
        def _():
            pltpu.sync_copy(acc, out.at[q])
        plsc.subcore_barrier()


_sc_agg = pl.kernel(
    _sc_agg_body,
    out_type=jax.ShapeDtypeStruct((4, NP, 16), _f32),
    mesh=_mesh(),
    compiler_params=pltpu.CompilerParams(use_tc_tiling_on_sc=False),
    scratch_types=[
        pltpu.VMEM_SHARED((NP, 16), _f32),
    ] + [pltpu.VMEM((CH,), jnp.int32) for _ in range(2 * NB)]
      + [pltpu.VMEM((CH, 16), _f32) for _ in range(NB)]
      + [pltpu.SemaphoreType.DMA for _ in range(3)],
)


def _gate_pre(h, dg, ds, Wg, Ws, Wwt, bw):
    yg = dg * jnp.dot(h, Wg, preferred_element_type=_f32)
    ys = ds * jnp.dot(h, Ws, preferred_element_type=_f32)
    s = jax.nn.sigmoid(jnp.sum(h * Wwt, axis=1, keepdims=True) + bw)
    return yg, ys, s


def _store_quarters(ref, y):
    for q in range(4):
        ref[q, 0] = y[:, 16 * q:16 * (q + 1)]


def _tc_init_body(x_ref, degg_ref, degs_ref, W1_ref, b1_ref, Wg_ref, Ws_ref,
                  Wwt_ref, bw_ref, yg_ref, ys_ref, s_ref, dg_ref, ds_ref):
    h = x_ref[...] * W1_ref[...] + b1_ref[...]
    dg = lax.rsqrt(degg_ref[:, 0:1] + 1.0)
    ds = lax.rsqrt(degs_ref[:, 0:1] + 1.0)
    yg, ys, s = _gate_pre(h, dg, ds, Wg_ref[...], Ws_ref[...], Wwt_ref[...],
                          bw_ref[...])
    _store_quarters(yg_ref, yg)
    _store_quarters(ys_ref, ys)
    s_ref[...] = s
    dg_ref[...] = dg
    ds_ref[...] = ds


def _combine(ag, asm, yg, ys, s_ref, dg_ref, ds_ref, bg_ref, bs_ref):
    dg = dg_ref[...]
    ds = ds_ref[...]
    agg_g = jnp.concatenate([r[...] for r in ag], axis=1)
    agg_s = jnp.concatenate([r[...] for r in asm], axis=1)
    y_g = jnp.concatenate([r[...] for r in yg], axis=1)
    y_s = jnp.concatenate([r[...] for r in ys], axis=1)
    xg = jnp.maximum(dg * (agg_g + y_g) + bg_ref[...], 0.0)
    xs = jnp.maximum(ds * (agg_s + y_s) + bs_ref[...], 0.0)
    s = s_ref[...]
    return s * xg + (1.0 - s) * xs, dg, ds


def _tc_mid_body(*refs):
    (ag0, ag1, ag2, ag3, as0, as1, as2, as3,
     yg0, yg1, yg2, yg3, ys0, ys1, ys2, ys3,
     s_ref, dg_ref, ds_ref, bg_ref, bs_ref,
     Wg_ref, Ws_ref, Wwt_ref, bw_ref, yg_ref, ys_ref, so_ref) = refs
    h, dg, ds = _combine((ag0, ag1, ag2, ag3), (as0, as1, as2, as3),
                         (yg0, yg1, yg2, yg3), (ys0, ys1, ys2, ys3),
                         s_ref, dg_ref, ds_ref, bg_ref, bs_ref)
    yg, ys, s = _gate_pre(h, dg, ds, Wg_ref[...], Ws_ref[...], Wwt_ref[...],
                          bw_ref[...])
    _store_quarters(yg_ref, yg)
    _store_quarters(ys_ref, ys)
    so_ref[...] = s


def _tc_fin_body(*refs):
    (ag0, ag1, ag2, ag3, as0, as1, as2, as3,
     yg0, yg1, yg2, yg3, ys0, ys1, ys2, ys3,
     s_ref, dg_ref, ds_ref, bg_ref, bs_ref, batch_ref, out_ref) = refs
    h, _, _ = _combine((ag0, ag1, ag2, ag3), (as0, as1, as2, as3),
                       (yg0, yg1, yg2, yg3), (ys0, ys1, ys2, ys3),
                       s_ref, dg_ref, ds_ref, bg_ref, bs_ref)
    i = pl.program_id(0)

    @pl.when(i == 0)
    def _():
        out_ref[...] = jnp.full((NG, H), -jnp.inf, _f32)

    bb = batch_ref[0, 0, :].reshape(R, 1)
    glo = jnp.min(bb)
    ghi = jnp.max(bb)

    def gbody(g, _):
        m = jnp.max(jnp.where(bb == g, h, -jnp.inf), axis=0)
        out_ref[pl.ds(g, 1), :] = jnp.maximum(out_ref[pl.ds(g, 1), :],
                                              m[None, :])
        return 0
    lax.fori_loop(glo, ghi + 1, gbody, 0)


def _row(i):
    return (i, 0)


_BS_X = pl.BlockSpec((R, 1), _row)
_BS_DEG = pl.BlockSpec((R, 16), _row)
_BS_16 = pl.BlockSpec((R, 16), _row)
_BS_1 = pl.BlockSpec((R, 1), _row)
_BS_Y4 = pl.BlockSpec((4, 1, R, 16), lambda i: (0, i, 0, 0))
_BS_W = pl.BlockSpec((H, H), lambda i: (0, 0))
_BS_W1 = pl.BlockSpec((1, H), lambda i: (0, 0))
_BS_BW = pl.BlockSpec((1, 1), lambda i: (0, 0))
_BS_BATCH = pl.BlockSpec((1, 1, R), lambda i: (i, 0, 0))
_BS_OUT = pl.BlockSpec((NG, H), lambda i: (0, 0))

_y4_shape = jax.ShapeDtypeStruct((4, GRID, R, 16), _f32)
_s_shape = jax.ShapeDtypeStruct((N, 1), _f32)

_tc_init = pl.pallas_call(
    _tc_init_body,
    grid=(GRID,),
    in_specs=[_BS_X, _BS_DEG, _BS_DEG, _BS_W1, _BS_W1, _BS_W, _BS_W,
              _BS_W1, _BS_BW],
    out_specs=[_BS_Y4, _BS_Y4, _BS_1, _BS_1, _BS_1],
    out_shape=[_y4_shape, _y4_shape, _s_shape, _s_shape, _s_shape],
)

_tc_mid = pl.pallas_call(
    _tc_mid_body,
    grid=(GRID,),
    in_specs=[_BS_16] * 16 + [_BS_1] * 3 + [_BS_W1, _BS_W1] +
             [_BS_W, _BS_W, _BS_W1, _BS_BW],
    out_specs=[_BS_Y4, _BS_Y4, _BS_1],
    out_shape=[_y4_shape, _y4_shape, _s_shape],
)

_tc_fin = pl.pallas_call(
    _tc_fin_body,
    grid=(GRID,),
    in_specs=[_BS_16] * 16 + [_BS_1] * 3 + [_BS_W1, _BS_W1] + [_BS_BATCH],
    out_specs=_BS_OUT,
    out_shape=jax.ShapeDtypeStruct((NG, H), _f32),
)


def _quarters(a4):
    return tuple(a4[q, :N] for q in range(4))


def kernel(x, edge_index, edge_index_sim, batch, W1, b1, Wg, bg, Ws, bs,
           Ww, bw):
    ei = edge_index.astype(jnp.int32)
    es = edge_index_sim.astype(jnp.int32)
    src_g, dst_g = ei[0], ei[1]
    src_s, dst_s = es[0], es[1]
    dstB = jnp.concatenate([dst_g, dst_s])
    gidx_g = jnp.concatenate([src_g, src_g + N, src_g + 2 * N,
                              src_g + 3 * N])
    gidx_s = jnp.concatenate([src_s, src_s + N, src_s + 2 * N,
                              src_s + 3 * N])
    batch3 = batch.astype(jnp.int32).reshape(GRID, 1, R)

    zeros16 = jnp.zeros((NP, 16), _f32)
    deg2 = _sc_deg(dstB, zeros16)
    degg = deg2[0, :N]
    degs = deg2[1, :N]
    b1r = b1.reshape(1, H)
    yg4, ys4, s, dg, ds = _tc_init(
        x, degg, degs, W1, b1r, Wg[0], Ws[0], Ww[0].reshape(1, H),
        bw[0].reshape(1, 1))

    for i in range(3):
        aggg = _sc_agg(yg4.reshape(4 * N, 16), gidx_g, dst_g, zeros16)
        aggs = _sc_agg(ys4.reshape(4 * N, 16), gidx_s, dst_s, zeros16)
        ygq = tuple(yg4.reshape(4, N, 16)[q] for q in range(4))
        ysq = tuple(ys4.reshape(4, N, 16)[q] for q in range(4))
        bgr = bg[i].reshape(1, H)
        bsr = bs[i].reshape(1, H)
        if i < 2:
            yg4, ys4, s = _tc_mid(
                *_quarters(aggg), *_quarters(aggs), *ygq, *ysq,
                s, dg, ds, bgr, bsr,
                Wg[i + 1], Ws[i + 1], Ww[i + 1].reshape(1, H),
                bw[i + 1].reshape(1, 1))
        else:
            out = _tc_fin(
                *_quarters(aggg), *_quarters(aggs), *ygq, *ysq,
                s, dg, ds, bgr, bsr, batch3)
    return out


# speedup vs baseline: 17.2862x; 1.1685x over previous
import jax
import jax.numpy as jnp
from jax import lax
from jax.experimental import pallas as pl
from jax.experimental.pallas import tpu as pltpu
from jax.experimental.pallas import tpu_sc as plsc

N = 50000
E = 800000
H = 64
NG = 64
R = 1000
GRID = N // R

NS = 16
EPT = E // NS
CH = 400
NB = 4
NGRP = 31
NTAIL = 1
RPT = 3128
NP = NS * RPT

_f32 = jnp.float32


def _mesh():
    return plsc.VectorSubcoreMesh(core_axis_name="c", subcore_axis_name="s")


def _sc_deg_body(dstB, zeros_hbm, out, acc, ones, *rest):
    c = lax.axis_index("c")
    t = lax.axis_index("s")
    dbufs = rest[:NB]
    sem_i, sem_s = rest[NB], rest[NB + 1]

    def fill1(i, _):
        ones[i, pl.ds(0, 16)] = jnp.ones((16,), _f32)
        return 0
    lax.fori_loop(0, CH, fill1, 0)

    pltpu.sync_copy(zeros_hbm.at[pl.ds(t * RPT, RPT)],
                    acc.at[pl.ds(t * RPT, RPT)])
    plsc.subcore_barrier()

    base = c * E + t * EPT

    def stage(j0, nb):
        ids = []
        for b in range(nb):
            off = base + j0 * CH + b * CH
            ids.append(pltpu.async_copy(dstB.at[pl.ds(off, CH)],
                                        dbufs[b], sem_i))
        for b in range(nb):
            ids[b].wait()
        sds = []
        for b in range(nb):
            sds.append(pltpu.async_copy(ones, acc.at[dbufs[b]], sem_s,
                                        add=True))
        for b in range(nb):
            sds[b].wait()

    def grp(g, _):
        stage(g * NB, NB)
        return 0
    lax.fori_loop(0, NGRP, grp, 0)
    stage(NGRP * NB, NTAIL)
    plsc.subcore_barrier()

    @pl.when(t == 0)
    def _():
        pltpu.sync_copy(acc, out.at[c])


_sc_deg = pl.kernel(
    _sc_deg_body,
    out_type=jax.ShapeDtypeStruct((2, NP, 16), _f32),
    mesh=_mesh(),
    compiler_params=pltpu.CompilerParams(use_tc_tiling_on_sc=False),
    scratch_types=[
        pltpu.VMEM_SHARED((NP, 16), _f32),
        pltpu.VMEM((CH, 16), _f32),
    ] + [pltpu.VMEM((CH,), jnp.int32) for _ in range(NB)]
      + [pltpu.SemaphoreType.DMA for _ in range(2)],
)


def _sc_agg_body(y4, gidx, dst, zeros_hbm, out, acc, *rest):
    c = lax.axis_index("c")
    t = lax.axis_index("s")
    sbufs = rest[:NB]
    dbufs = rest[NB:2 * NB]
    rbufs = rest[2 * NB:3 * NB]
    sem_i, sem_g, sem_s = rest[3 * NB:3 * NB + 3]

    for p in range(2):
        q = 2 * c + p
        pltpu.sync_copy(zeros_hbm.at[pl.ds(t * RPT, RPT)],
                        acc.at[pl.ds(t * RPT, RPT)])
        plsc.subcore_barrier()

        base = q * E + t * EPT
        dbase = t * EPT

        def stage(j0, nb):
            ids = []
            for b in range(nb):
                i1 = pltpu.async_copy(gidx.at[pl.ds(base + j0 * CH + b * CH,
                                                    CH)], sbufs[b], sem_i)
                i2 = pltpu.async_copy(dst.at[pl.ds(dbase + j0 * CH + b * CH,
                                                   CH)], dbufs[b], sem_i)
                ids.append((i1, i2))
            for b in range(nb):
                ids[b][0].wait()
                ids[b][1].wait()
            gds = []
            for b in range(nb):
                gds.append(pltpu.async_copy(y4.at[sbufs[b]], rbufs[b],
                                            sem_g))
            for b in range(nb):
                gds[b].wait()
            sds = []
            for b in range(nb):
                sds.append(pltpu.async_copy(rbufs[b], acc.at[dbufs[b]],
                                            sem_s, add=True))
            for b in range(nb):
                sds[b].wait()

        def grp(g, _):
            stage(g * NB, NB)
            return 0
        lax.fori_loop(0, NGRP, grp, 0)
        stage(NGRP * NB, NTAIL)
        plsc.subcore_barrier()

        @pl.when(t == 0)
        def _():
            pltpu.sync_copy(acc, out.at[q])
        plsc.subcore_barrier()


_sc_agg = pl.kernel(
    _sc_agg_body,
    out_type=jax.ShapeDtypeStruct((4, NP, 16), _f32),
    mesh=_mesh(),
    compiler_params=pltpu.CompilerParams(use_tc_tiling_on_sc=False),
    scratch_types=[
        pltpu.VMEM_SHARED((NP, 16), _f32),
    ] + [pltpu.VMEM((CH,), jnp.int32) for _ in range(2 * NB)]
      + [pltpu.VMEM((CH, 16), _f32) for _ in range(NB)]
      + [pltpu.SemaphoreType.DMA for _ in range(3)],
)


def _gate_pre(h, dg, ds, Wg, Ws, Wwt, bw):
    yg = dg * jnp.dot(h, Wg, preferred_element_type=_f32)
    ys = ds * jnp.dot(h, Ws, preferred_element_type=_f32)
    s = jax.nn.sigmoid(jnp.sum(h * Wwt, axis=1, keepdims=True) + bw)
    return yg, ys, s


def _store_quarters(ref, y):
    for q in range(4):
        ref[q, 0] = y[:, 16 * q:16 * (q + 1)]


def _tc_init_body(x_ref, degg_ref, degs_ref, W1_ref, b1_ref, Wg_ref, Ws_ref,
                  Wwt_ref, bw_ref, yg_ref, ys_ref, s_ref, dg_ref, ds_ref):
    h = x_ref[...] * W1_ref[...] + b1_ref[...]
    dg = lax.rsqrt(degg_ref[0, :, 0:1] + 1.0)
    ds = lax.rsqrt(degs_ref[0, :, 0:1] + 1.0)
    yg, ys, s = _gate_pre(h, dg, ds, Wg_ref[...], Ws_ref[...], Wwt_ref[...],
                          bw_ref[...])
    _store_quarters(yg_ref, yg)
    _store_quarters(ys_ref, ys)
    s_ref[...] = s
    dg_ref[...] = dg
    ds_ref[...] = ds


def _combine(ag, asm, yg, ys, s_ref, dg_ref, ds_ref, bg_ref, bs_ref):
    dg = dg_ref[...]
    ds = ds_ref[...]
    agg_g = jnp.concatenate([r[0] for r in ag], axis=1)
    agg_s = jnp.concatenate([r[0] for r in asm], axis=1)
    y_g = jnp.concatenate([r[0, 0] for r in yg], axis=1)
    y_s = jnp.concatenate([r[0, 0] for r in ys], axis=1)
    xg = jnp.maximum(dg * (agg_g + y_g) + bg_ref[...], 0.0)
    xs = jnp.maximum(ds * (agg_s + y_s) + bs_ref[...], 0.0)
    s = s_ref[...]
    return s * xg + (1.0 - s) * xs, dg, ds


def _tc_mid_body(*refs):
    (ag0, ag1, ag2, ag3, as0, as1, as2, as3,
     yg0, yg1, yg2, yg3, ys0, ys1, ys2, ys3,
     s_ref, dg_ref, ds_ref, bg_ref, bs_ref,
     Wg_ref, Ws_ref, Wwt_ref, bw_ref, yg_ref, ys_ref, so_ref) = refs
    h, dg, ds = _combine((ag0, ag1, ag2, ag3), (as0, as1, as2, as3),
                         (yg0, yg1, yg2, yg3), (ys0, ys1, ys2, ys3),
                         s_ref, dg_ref, ds_ref, bg_ref, bs_ref)
    yg, ys, s = _gate_pre(h, dg, ds, Wg_ref[...], Ws_ref[...], Wwt_ref[...],
                          bw_ref[...])
    _store_quarters(yg_ref, yg)
    _store_quarters(ys_ref, ys)
    so_ref[...] = s


def _tc_fin_body(*refs):
    (ag0, ag1, ag2, ag3, as0, as1, as2, as3,
     yg0, yg1, yg2, yg3, ys0, ys1, ys2, ys3,
     s_ref, dg_ref, ds_ref, bg_ref, bs_ref, batch_ref, out_ref) = refs
    h, _, _ = _combine((ag0, ag1, ag2, ag3), (as0, as1, as2, as3),
                       (yg0, yg1, yg2, yg3), (ys0, ys1, ys2, ys3),
                       s_ref, dg_ref, ds_ref, bg_ref, bs_ref)
    i = pl.program_id(0)

    @pl.when(i == 0)
    def _():
        out_ref[...] = jnp.full((NG, H), -jnp.inf, _f32)

    bb = batch_ref[0, 0, :].reshape(R, 1)
    glo = jnp.min(bb)
    ghi = jnp.max(bb)

    def gbody(g, _):
        m = jnp.max(jnp.where(bb == g, h, -jnp.inf), axis=0)
        out_ref[pl.ds(g, 1), :] = jnp.maximum(out_ref[pl.ds(g, 1), :],
                                              m[None, :])
        return 0
    lax.fori_loop(glo, ghi + 1, gbody, 0)


def _row(i):
    return (i, 0)


_BS_X = pl.BlockSpec((R, 1), _row)
_BS_1 = pl.BlockSpec((R, 1), _row)


def _bs_aq(q):
    return pl.BlockSpec((1, R, 16), lambda i, q=q: (q, i, 0))


def _bs_yq(q):
    return pl.BlockSpec((1, 1, R, 16), lambda i, q=q: (q, i, 0, 0))


_BS_AQ = [_bs_aq(q) for q in range(4)]
_BS_YQ = [_bs_yq(q) for q in range(4)]
_BS_Y4 = pl.BlockSpec((4, 1, R, 16), lambda i: (0, i, 0, 0))
_BS_W = pl.BlockSpec((H, H), lambda i: (0, 0))
_BS_W1 = pl.BlockSpec((1, H), lambda i: (0, 0))
_BS_BW = pl.BlockSpec((1, 1), lambda i: (0, 0))
_BS_BATCH = pl.BlockSpec((1, 1, R), lambda i: (i, 0, 0))
_BS_OUT = pl.BlockSpec((NG, H), lambda i: (0, 0))

_y4_shape = jax.ShapeDtypeStruct((4, GRID, R, 16), _f32)
_s_shape = jax.ShapeDtypeStruct((N, 1), _f32)

_tc_init = pl.pallas_call(
    _tc_init_body,
    grid=(GRID,),
    in_specs=[_BS_X, _bs_aq(0), _bs_aq(1), _BS_W1, _BS_W1, _BS_W, _BS_W,
              _BS_W1, _BS_BW],
    out_specs=[_BS_Y4, _BS_Y4, _BS_1, _BS_1, _BS_1],
    out_shape=[_y4_shape, _y4_shape, _s_shape, _s_shape, _s_shape],
)

_tc_mid = pl.pallas_call(
    _tc_mid_body,
    grid=(GRID,),
    in_specs=_BS_AQ + _BS_AQ + _BS_YQ + _BS_YQ + [_BS_1] * 3 +
             [_BS_W1, _BS_W1] + [_BS_W, _BS_W, _BS_W1, _BS_BW],
    out_specs=[_BS_Y4, _BS_Y4, _BS_1],
    out_shape=[_y4_shape, _y4_shape, _s_shape],
)

_tc_fin = pl.pallas_call(
    _tc_fin_body,
    grid=(GRID,),
    in_specs=_BS_AQ + _BS_AQ + _BS_YQ + _BS_YQ + [_BS_1] * 3 +
             [_BS_W1, _BS_W1] + [_BS_BATCH],
    out_specs=_BS_OUT,
    out_shape=jax.ShapeDtypeStruct((NG, H), _f32),
)


def kernel(x, edge_index, edge_index_sim, batch, W1, b1, Wg, bg, Ws, bs,
           Ww, bw):
    ei = edge_index.astype(jnp.int32)
    es = edge_index_sim.astype(jnp.int32)
    src_g, dst_g = ei[0], ei[1]
    src_s, dst_s = es[0], es[1]
    dstB = jnp.concatenate([dst_g, dst_s])
    gidx_g = jnp.concatenate([src_g, src_g + N, src_g + 2 * N,
                              src_g + 3 * N])
    gidx_s = jnp.concatenate([src_s, src_s + N, src_s + 2 * N,
                              src_s + 3 * N])
    batch3 = batch.astype(jnp.int32).reshape(GRID, 1, R)

    zeros16 = jnp.zeros((NP, 16), _f32)
    deg2 = _sc_deg(dstB, zeros16)
    b1r = b1.reshape(1, H)
    yg4, ys4, s, dg, ds = _tc_init(
        x, deg2, deg2, W1, b1r, Wg[0], Ws[0], Ww[0].reshape(1, H),
        bw[0].reshape(1, 1))

    for i in range(3):
        aggg = _sc_agg(yg4.reshape(4 * N, 16), gidx_g, dst_g, zeros16)
        aggs = _sc_agg(ys4.reshape(4 * N, 16), gidx_s, dst_s, zeros16)
        bgr = bg[i].reshape(1, H)
        bsr = bs[i].reshape(1, H)
        if i < 2:
            yg4, ys4, s = _tc_mid(
                aggg, aggg, aggg, aggg, aggs, aggs, aggs, aggs,
                yg4, yg4, yg4, yg4, ys4, ys4, ys4, ys4,
                s, dg, ds, bgr, bsr,
                Wg[i + 1], Ws[i + 1], Ww[i + 1].reshape(1, H),
                bw[i + 1].reshape(1, 1))
        else:
            out = _tc_fin(
                aggg, aggg, aggg, aggg, aggs, aggs, aggs, aggs,
                yg4, yg4, yg4, yg4, ys4, ys4, ys4, ys4,
                s, dg, ds, bgr, bsr, batch3)
    return out
